# Initial kernel scaffold; baseline (speedup 1.0000x reference)
import jax
import jax.numpy as jnp
from jax.experimental import pallas as pl


def kernel(gt_labels_list, gt_boxes_list, label_encoder_weight):
    raise NotImplementedError("write your pallas kernel here")


# speedup vs baseline: 2.9393x; 2.9393x over previous
import functools

import numpy as np
import jax
import jax.numpy as jnp
from jax import lax
from jax.experimental import pallas as pl
from jax.experimental.pallas import tpu as pltpu
from jax.experimental.pallas import tpu_sc as plsc

_NUM_QUERIES = 900
_NUM_CLASSES = 365
_D = 256
_B = 64
_G = 500
_N = _B * _G
_LABEL_NOISE_PROB = 0.5

_ROWS_PER_W = 1000
_CHUNK = 40
_NCHUNK = _ROWS_PER_W // _CHUNK


def _tf_rotl(x, r):
    return (x << np.uint32(r)) | (x >> np.uint32(32 - r))


def _threefry2x32(ks0, ks1, x0, x1):
    rotations = ((13, 15, 26, 6), (17, 29, 16, 24))
    ks = (np.uint32(ks0), np.uint32(ks1),
          np.uint32(ks0) ^ np.uint32(ks1) ^ np.uint32(0x1BD11BDA))
    x0 = x0 + ks[0]
    x1 = x1 + ks[1]
    for i in range(5):
        for r in rotations[i % 2]:
            x0 = x0 + x1
            x1 = _tf_rotl(x1, r)
            x1 = x1 ^ x0
        x0 = x0 + ks[(i + 1) % 3]
        x1 = x1 + ks[(i + 2) % 3] + np.uint32(i + 1)
    return x0, x1


def _random_bits(key, size):
    o0, o1 = _threefry2x32(key[0], key[1], np.zeros(size, np.uint32),
                           np.arange(size, dtype=np.uint32))
    return o0 ^ o1


def _split_key(key, num):
    o0, o1 = _threefry2x32(key[0], key[1], np.zeros(num, np.uint32),
                           np.arange(num, dtype=np.uint32))
    return np.stack([o0, o1], axis=1)


def _uniform01(key, size):
    bits = _random_bits(key, size)
    f = ((bits >> np.uint32(9)) | np.uint32(0x3F800000)).view(np.float32)
    return np.maximum(np.float32(0.0), f - np.float32(1.0))


def _randint(key, size, span):
    k1, k2 = _split_key(key, 2)
    hi = _random_bits(k1, size)
    lo = _random_bits(k2, size)
    span = np.uint32(span)
    mult = np.uint32((int(2 ** 16 % int(span)) ** 2) % int(span))
    off = ((hi % span) * mult + (lo % span)) % span
    return off.astype(np.int32)


@functools.cache
def _noise_consts():
    kmask, klab, kbox = _split_key(np.array([0, 42], np.uint32), 3)
    mask = _uniform01(kmask, _N) < np.float32(_LABEL_NOISE_PROB)
    rand_labels = _randint(klab, _N, _NUM_CLASSES)
    rand_box = (_uniform01(kbox, _N * 4) * np.float32(2.0)
                - np.float32(1.0)).reshape(_N, 4)
    return (
        mask.astype(np.int32),
        rand_labels.astype(np.int32),
        rand_box.astype(np.float32),
    )


@functools.cache
def _attn_mask_const():
    m = np.zeros((_G + _NUM_QUERIES, _G + _NUM_QUERIES), dtype=bool)
    m[_G:, :_G] = True
    return m


def _tc_noise_body(box_ref, rbox_ref, lab_ref, mask_ref, rlab_ref,
                   obox_ref, olab_ref):
    b = box_ref[...]
    r = rbox_ref[...]
    shifted = jnp.concatenate([b[:, 2:], b[:, :2]], axis=1)
    lane = lax.broadcasted_iota(jnp.int32, b.shape, 1)
    diff = jnp.where((lane & 3) < 2, shifted * 0.5, b)
    nb = jnp.clip(b + r * diff, 0.0, 1.0)
    eps = 1e-5
    x1 = jnp.maximum(nb, eps)
    x2 = jnp.maximum(1.0 - nb, eps)
    obox_ref[...] = jnp.log(x1 / x2)
    olab_ref[...] = jnp.where(mask_ref[...] != 0, rlab_ref[...], lab_ref[...])


def _tc_noise(boxes2d, rbox2d, labels2d, mask2d, rlab2d):
    return pl.pallas_call(
        _tc_noise_body,
        out_shape=(
            jax.ShapeDtypeStruct((_N * 4 // 128, 128), jnp.float32),
            jax.ShapeDtypeStruct((_N // 128, 128), jnp.int32),
        ),
    )(boxes2d, rbox2d, labels2d, mask2d, rlab2d)


def _sc_gather(labels3d, table):
    info = plsc.get_sparse_core_info()
    nc = info.num_cores

    mesh = plsc.VectorSubcoreMesh(core_axis_name="c", subcore_axis_name="s")

    @functools.partial(
        pl.kernel,
        mesh=mesh,
        out_type=jax.ShapeDtypeStruct((_N, _D), jnp.float32),
        scratch_types=[
            pltpu.VMEM((_NCHUNK, _CHUNK), jnp.int32),
            pltpu.VMEM((_CHUNK, _D), jnp.float32),
            pltpu.VMEM((_CHUNK, _D), jnp.float32),
            pltpu.SemaphoreType.DMA,
            pltpu.SemaphoreType.DMA,
        ],
    )
    def k(lab_hbm, tab_hbm, out_hbm, idx_v, buf0, buf1, sem0, sem1):
        wid = lax.axis_index("s") * nc + lax.axis_index("c")
        base = wid * _ROWS_PER_W
        pltpu.sync_copy(lab_hbm.at[wid], idx_v)
        bufs = (buf0, buf1)
        sems = (sem0, sem1)
        copies = [pltpu.async_copy(tab_hbm.at[idx_v.at[0]], buf0, sem0)]
        for c in range(_NCHUNK):
            if c + 1 < _NCHUNK:
                copies.append(
                    pltpu.async_copy(
                        tab_hbm.at[idx_v.at[c + 1]],
                        bufs[(c + 1) % 2],
                        sems[(c + 1) % 2],
                    )
                )
            copies[c].wait()
            pltpu.sync_copy(bufs[c % 2],
                            out_hbm.at[pl.ds(base + c * _CHUNK, _CHUNK)])

    return k(labels3d, table)


def kernel(gt_labels_list, gt_boxes_list, label_encoder_weight):
    mask_np, rlab_np, rbox_np = _noise_consts()
    boxes2d = gt_boxes_list.astype(jnp.float32).reshape(_N * 4 // 128, 128)
    labels2d = gt_labels_list.astype(jnp.int32).reshape(_N // 128, 128)
    obox2d, olab2d = _tc_noise(
        boxes2d,
        jnp.asarray(rbox_np).reshape(_N * 4 // 128, 128),
        labels2d,
        jnp.asarray(mask_np).reshape(_N // 128, 128),
        jnp.asarray(rlab_np).reshape(_N // 128, 128),
    )
    lab3d = olab2d.reshape(_N // _ROWS_PER_W, _NCHUNK, _CHUNK)
    emb = _sc_gather(lab3d, label_encoder_weight.astype(jnp.float32))
    noised_label_queries = emb.reshape(_B, _G, _D)
    noised_box_queries = obox2d.reshape(_B, _G, 4)
    attn_mask = jnp.asarray(_attn_mask_const())
    return (noised_label_queries, noised_box_queries, attn_mask, 1, _G)


# speedup vs baseline: 3.2256x; 1.0974x over previous
import functools

import numpy as np
import jax
import jax.numpy as jnp
from jax import lax
from jax.experimental import pallas as pl
from jax.experimental.pallas import tpu as pltpu
from jax.experimental.pallas import tpu_sc as plsc

_NUM_QUERIES = 900
_NUM_CLASSES = 365
_D = 256
_B = 64
_G = 500
_N = _B * _G
_LABEL_NOISE_PROB = 0.5

_ROWS_PER_W = 1000
_CHUNK = 40
_NCHUNK = _ROWS_PER_W // _CHUNK


def _tf_rotl(x, r):
    return (x << np.uint32(r)) | (x >> np.uint32(32 - r))


def _threefry2x32(ks0, ks1, x0, x1):
    rotations = ((13, 15, 26, 6), (17, 29, 16, 24))
    ks = (np.uint32(ks0), np.uint32(ks1),
          np.uint32(ks0) ^ np.uint32(ks1) ^ np.uint32(0x1BD11BDA))
    x0 = x0 + ks[0]
    x1 = x1 + ks[1]
    for i in range(5):
        for r in rotations[i % 2]:
            x0 = x0 + x1
            x1 = _tf_rotl(x1, r)
            x1 = x1 ^ x0
        x0 = x0 + ks[(i + 1) % 3]
        x1 = x1 + ks[(i + 2) % 3] + np.uint32(i + 1)
    return x0, x1


def _random_bits(key, size):
    o0, o1 = _threefry2x32(key[0], key[1], np.zeros(size, np.uint32),
                           np.arange(size, dtype=np.uint32))
    return o0 ^ o1


def _split_key(key, num):
    o0, o1 = _threefry2x32(key[0], key[1], np.zeros(num, np.uint32),
                           np.arange(num, dtype=np.uint32))
    return np.stack([o0, o1], axis=1)


def _uniform01(key, size):
    bits = _random_bits(key, size)
    f = ((bits >> np.uint32(9)) | np.uint32(0x3F800000)).view(np.float32)
    return np.maximum(np.float32(0.0), f - np.float32(1.0))


def _randint(key, size, span):
    k1, k2 = _split_key(key, 2)
    hi = _random_bits(k1, size)
    lo = _random_bits(k2, size)
    span = np.uint32(span)
    mult = np.uint32((int(2 ** 16 % int(span)) ** 2) % int(span))
    off = ((hi % span) * mult + (lo % span)) % span
    return off.astype(np.int32)


@functools.cache
def _noise_consts():
    kmask, klab, kbox = _split_key(np.array([0, 42], np.uint32), 3)
    mask = _uniform01(kmask, _N) < np.float32(_LABEL_NOISE_PROB)
    rand_labels = _randint(klab, _N, _NUM_CLASSES)
    rand_box = (_uniform01(kbox, _N * 4) * np.float32(2.0)
                - np.float32(1.0)).reshape(_N, 4)
    return (
        mask.astype(np.int32),
        rand_labels.astype(np.int32),
        rand_box.astype(np.float32),
    )


@functools.cache
def _attn_mask_const():
    m = np.zeros((_G + _NUM_QUERIES, _G + _NUM_QUERIES), dtype=bool)
    m[_G:, :_G] = True
    return m


def _tc_noise_body(box_ref, rbox_ref, lab_ref, mask_ref, rlab_ref,
                   obox_ref, olab_ref):
    b = box_ref[...]
    r = rbox_ref[...]
    shifted = jnp.concatenate([b[:, 2:], b[:, :2]], axis=1)
    lane = lax.broadcasted_iota(jnp.int32, b.shape, 1)
    diff = jnp.where((lane & 3) < 2, shifted * 0.5, b)
    nb = jnp.clip(b + r * diff, 0.0, 1.0)
    eps = 1e-5
    x1 = jnp.maximum(nb, eps)
    x2 = jnp.maximum(1.0 - nb, eps)
    obox_ref[...] = jnp.log(x1 / x2)
    olab_ref[...] = jnp.where(mask_ref[...] != 0, rlab_ref[...], lab_ref[...])


def _tc_noise(boxes2d, rbox2d, labels2d, mask2d, rlab2d):
    return pl.pallas_call(
        _tc_noise_body,
        out_shape=(
            jax.ShapeDtypeStruct((_N * 4 // 128, 128), jnp.float32),
            jax.ShapeDtypeStruct((_N // 128, 128), jnp.int32),
        ),
    )(boxes2d, rbox2d, labels2d, mask2d, rlab2d)


_BATCH_CHUNKS = tuple((c * _CHUNK, _CHUNK) for c in range(12)) + ((480, 16),)


def _sc_gather(labels_flat, table):
    info = plsc.get_sparse_core_info()
    nc = info.num_cores

    mesh = plsc.VectorSubcoreMesh(core_axis_name="c", subcore_axis_name="s")

    @functools.partial(
        pl.kernel,
        mesh=mesh,
        out_type=(jax.ShapeDtypeStruct((_B, _G, _D), jnp.float32),
                  jax.ShapeDtypeStruct((_B, 8, _D), jnp.float32)),
        scratch_types=[
            pltpu.VMEM((1024,), jnp.int32),
            pltpu.VMEM((_CHUNK, _D), jnp.float32),
            pltpu.VMEM((_CHUNK, _D), jnp.float32),
            pltpu.SemaphoreType.DMA,
            pltpu.SemaphoreType.DMA,
        ],
    )
    def k(lab_hbm, tab_hbm, out_hbm, tails_hbm, idx_v, buf0, buf1,
          sem0, sem1):
        wid = lax.axis_index("s") * nc + lax.axis_index("c")
        b0 = wid * 2
        pltpu.sync_copy(lab_hbm.at[pl.ds(wid * 1024, 1024)], idx_v)
        bufs = (buf0, buf1)
        sems = (sem0, sem1)
        sched = []
        for i in (0, 1):
            sched += [(512 * i + off, n,
                       out_hbm.at[b0 + i].at[pl.ds(off, n)])
                      for off, n in _BATCH_CHUNKS]
            sched.append((512 * i + 496, 8, tails_hbm.at[b0 + i]))
        copies = []
        ioff, n = sched[0][:2]
        copies.append(
            pltpu.async_copy(tab_hbm.at[idx_v.at[pl.ds(ioff, n)]],
                             buf0.at[pl.ds(0, n)], sem0))
        for c, (_, n, dst) in enumerate(sched):
            if c + 1 < len(sched):
                nioff, nn = sched[c + 1][:2]
                copies.append(
                    pltpu.async_copy(
                        tab_hbm.at[idx_v.at[pl.ds(nioff, nn)]],
                        bufs[(c + 1) % 2].at[pl.ds(0, nn)],
                        sems[(c + 1) % 2],
                    )
                )
            copies[c].wait()
            pltpu.sync_copy(bufs[c % 2].at[pl.ds(0, n)], dst)

    return k(labels_flat, table)


def kernel(gt_labels_list, gt_boxes_list, label_encoder_weight):
    mask_np, rlab_np, rbox_np = _noise_consts()
    boxes2d = gt_boxes_list.astype(jnp.float32).reshape(_N * 4 // 128, 128)
    labels2d = gt_labels_list.astype(jnp.int32).reshape(_N // 128, 128)
    obox2d, olab2d = _tc_noise(
        boxes2d,
        jnp.asarray(rbox_np).reshape(_N * 4 // 128, 128),
        labels2d,
        jnp.asarray(mask_np).reshape(_N // 128, 128),
        jnp.asarray(rlab_np).reshape(_N // 128, 128),
    )
    lab_flat = jnp.pad(olab2d.reshape(_B, _G), ((0, 0), (0, 12))).reshape(-1)
    main, tails = _sc_gather(lab_flat,
                             label_encoder_weight.astype(jnp.float32))
    noised_label_queries = lax.dynamic_update_slice(
        main, tails[:, :4, :], (0, _G - 4, 0))
    noised_box_queries = obox2d.reshape(_B, _G, 4)
    attn_mask = jnp.asarray(_attn_mask_const())
    return (noised_label_queries, noised_box_queries, attn_mask, 1, _G)
